# baseline (speedup 1.0000x reference)
import functools

import jax
import jax.numpy as jnp
from jax import lax
from jax.experimental import pallas as pl
from jax.experimental.pallas import tpu as pltpu
from jax.experimental.pallas import tpu_sc as plsc

_N_E = 8192
_E_DIM = 256
_BETA = 0.25
_M = 9216
_BM = 384
_MT = _M // _BM


def _en_body(e_ref, en_ref):
    e = e_ref[...]
    en_ref[...] = jnp.sum(e * e, axis=1, keepdims=True).T


_en_call = pl.pallas_call(
    _en_body,
    out_shape=jax.ShapeDtypeStruct((1, _N_E), jnp.float32),
)


def _vq_body(z_ref, e_ref, en_ref, idx_ref, oh_ref, dmin_ref, idx_s):
    j = pl.program_id(1)

    @pl.when(j == 0)
    def _():
        z = z_ref[...]
        zn = jnp.sum(z * z, axis=1, keepdims=True)
        en = en_ref[...]
        mm = lax.dot_general(z, e_ref[...], (((1,), (1,)), ((), ())),
                             preferred_element_type=jnp.float32)
        d = (zn + en) - 2.0 * mm
        dmin = jnp.min(d, axis=1, keepdims=True)
        col = lax.broadcasted_iota(jnp.int32, d.shape, 1)
        idx = jnp.min(jnp.where(d == dmin, col, _N_E), axis=1, keepdims=True)
        idx_s[...] = idx
        idx_ref[...] = idx
        dmin_ref[...] = dmin
        colh = lax.broadcasted_iota(jnp.int32, (_BM, _N_E // 2), 1)
        oh_ref[...] = (colh == idx).astype(jnp.float32)

    @pl.when(j == 1)
    def _():
        idx = idx_s[...]
        colh = lax.broadcasted_iota(jnp.int32, (_BM, _N_E // 2), 1) + _N_E // 2
        oh_ref[...] = (colh == idx).astype(jnp.float32)


_vq_call = pl.pallas_call(
    _vq_body,
    grid=(_MT, 2),
    in_specs=[
        pl.BlockSpec((_BM, _E_DIM), lambda i, j: (i, 0)),
        pl.BlockSpec((_N_E, _E_DIM), lambda i, j: (0, 0)),
        pl.BlockSpec((1, _N_E), lambda i, j: (0, 0)),
    ],
    out_specs=[
        pl.BlockSpec((_BM, 1), lambda i, j: (i, 0)),
        pl.BlockSpec((_BM, _N_E // 2), lambda i, j: (i, j)),
        pl.BlockSpec((_BM, 1), lambda i, j: (i, 0)),
    ],
    out_shape=[
        jax.ShapeDtypeStruct((_M, 1), jnp.int32),
        jax.ShapeDtypeStruct((_M, _N_E), jnp.float32),
        jax.ShapeDtypeStruct((_M, 1), jnp.float32),
    ],
    scratch_shapes=[
        pltpu.VMEM((_BM, 1), jnp.int32),
    ],
)


def _fin_body(cnt2_ref, dmin_ref, loss_ref, ppl_ref):
    lsum = jnp.sum(dmin_ref[...], axis=0, keepdims=True)
    loss_ref[...] = lsum * ((1.0 + _BETA) / (_M * _E_DIM))
    counts = cnt2_ref[0:1, :] + cnt2_ref[1:2, :]
    e_mean = counts / jnp.float32(_M)
    ent = jnp.sum(e_mean * jnp.log(e_mean + 1e-10), axis=1, keepdims=True)
    ppl_ref[...] = jnp.exp(-ent)


_fin_call = pl.pallas_call(
    _fin_body,
    out_shape=[
        jax.ShapeDtypeStruct((1, 1), jnp.float32),
        jax.ShapeDtypeStruct((1, 1), jnp.float32),
    ],
)


_NC = 2
_NS = 16
_NW = _NC * _NS
_BPW = _M // _NW
_SL = _N_E // _NS


@functools.partial(
    pl.kernel,
    mesh=plsc.VectorSubcoreMesh(core_axis_name="c", subcore_axis_name="s"),
    out_type=[
        jax.ShapeDtypeStruct((_M, _E_DIM), jnp.float32),
        jax.ShapeDtypeStruct((_NC, _N_E), jnp.float32),
    ],
    scratch_types=[
        pltpu.VMEM((_BPW,), jnp.int32),
        pltpu.VMEM((_BPW, _E_DIM), jnp.float32),
        pltpu.VMEM((_BPW,), jnp.float32),
        pltpu.VMEM((_SL,), jnp.float32),
        pltpu.VMEM_SHARED((_N_E,), jnp.float32),
        pltpu.SemaphoreType.DMA,
    ],
)
def _sc_gather(table_hbm, idx_hbm, out_hbm, cnt_hbm,
               idx_v, rows_v, ones_v, zero_v, hist_sh, sem):
    c = lax.axis_index("c")
    s = lax.axis_index("s")
    wid = s * _NC + c
    base = wid * _BPW
    pltpu.sync_copy(idx_hbm.at[pl.ds(base, _BPW)], idx_v)
    pltpu.async_copy(table_hbm.at[idx_v], rows_v, sem).wait()
    pltpu.sync_copy(rows_v, out_hbm.at[pl.ds(base, _BPW)])

    for k in range(_BPW // 16):
        ones_v[pl.ds(k * 16, 16)] = jnp.full((16,), 1.0, jnp.float32)
    for k in range(_SL // 16):
        zero_v[pl.ds(k * 16, 16)] = jnp.zeros((16,), jnp.float32)
    pltpu.sync_copy(zero_v, hist_sh.at[pl.ds(s * _SL, _SL)])
    plsc.subcore_barrier()
    pltpu.sync_copy(ones_v, hist_sh.at[idx_v], add=True)
    plsc.subcore_barrier()
    pltpu.sync_copy(hist_sh.at[pl.ds(s * _SL, _SL)],
                    cnt_hbm.at[c, pl.ds(s * _SL, _SL)])


def kernel(z, embedding_weight):
    zf = z.reshape(-1, _E_DIM)
    en = _en_call(embedding_weight)
    idx, oh, dmin = _vq_call(zf, embedding_weight, en)
    zq, cnt2 = _sc_gather(embedding_weight, idx.reshape(_M))
    loss, ppl = _fin_call(cnt2, dmin)
    return (loss[0, 0], zq.reshape(z.shape), ppl[0, 0], oh, idx)


# speedup vs baseline: 1.2813x; 1.2813x over previous
import functools

import jax
import jax.numpy as jnp
from jax import lax
from jax.experimental import pallas as pl
from jax.experimental.pallas import tpu as pltpu
from jax.experimental.pallas import tpu_sc as plsc

_N_E = 8192
_E_DIM = 256
_BETA = 0.25
_M = 9216
_BM = 384
_MT = _M // _BM


def _en_body(e_ref, en_ref):
    e = e_ref[...]
    en_ref[...] = jnp.sum(e * e, axis=1, keepdims=True).T


_en_call = pl.pallas_call(
    _en_body,
    out_shape=jax.ShapeDtypeStruct((1, _N_E), jnp.float32),
)


def _vq_body(z_ref, e_ref, en_ref, idx_ref, oh_ref, dmin_ref):
    z = z_ref[...]
    zn = jnp.sum(z * z, axis=1, keepdims=True)
    en = en_ref[...]
    mm = lax.dot_general(z, e_ref[...], (((1,), (1,)), ((), ())),
                         preferred_element_type=jnp.float32)
    d = (zn + en) - 2.0 * mm
    dmin = jnp.min(d, axis=1, keepdims=True)
    col = lax.broadcasted_iota(jnp.int32, d.shape, 1)
    idx = jnp.min(jnp.where(d == dmin, col, _N_E), axis=1, keepdims=True)
    oh_ref[...] = (col == idx).astype(jnp.float32)
    idx_ref[...] = idx
    dmin_ref[...] = dmin


_vq_call = pl.pallas_call(
    _vq_body,
    grid=(_MT,),
    in_specs=[
        pl.BlockSpec((_BM, _E_DIM), lambda i: (i, 0)),
        pl.BlockSpec((_N_E, _E_DIM), lambda i: (0, 0)),
        pl.BlockSpec((1, _N_E), lambda i: (0, 0)),
    ],
    out_specs=[
        pl.BlockSpec((_BM, 1), lambda i: (i, 0)),
        pl.BlockSpec((_BM, _N_E), lambda i: (i, 0)),
        pl.BlockSpec((_BM, 1), lambda i: (i, 0)),
    ],
    out_shape=[
        jax.ShapeDtypeStruct((_M, 1), jnp.int32),
        jax.ShapeDtypeStruct((_M, _N_E), jnp.float32),
        jax.ShapeDtypeStruct((_M, 1), jnp.float32),
    ],
)


def _fin_body(cnt2_ref, dmin_ref, loss_ref, ppl_ref):
    lsum = jnp.sum(dmin_ref[...], axis=0, keepdims=True)
    loss_ref[...] = lsum * ((1.0 + _BETA) / (_M * _E_DIM))
    counts = cnt2_ref[0:1, :] + cnt2_ref[1:2, :]
    e_mean = counts / jnp.float32(_M)
    ent = jnp.sum(e_mean * jnp.log(e_mean + 1e-10), axis=1, keepdims=True)
    ppl_ref[...] = jnp.exp(-ent)


_fin_call = pl.pallas_call(
    _fin_body,
    out_shape=[
        jax.ShapeDtypeStruct((1, 1), jnp.float32),
        jax.ShapeDtypeStruct((1, 1), jnp.float32),
    ],
)


_NC = 2
_NS = 16
_NW = _NC * _NS
_BPW = _M // _NW
_SL = _N_E // _NS


@functools.partial(
    pl.kernel,
    mesh=plsc.VectorSubcoreMesh(core_axis_name="c", subcore_axis_name="s"),
    out_type=[
        jax.ShapeDtypeStruct((_M, _E_DIM), jnp.float32),
        jax.ShapeDtypeStruct((_NC, _N_E), jnp.float32),
    ],
    scratch_types=[
        pltpu.VMEM((_BPW,), jnp.int32),
        pltpu.VMEM((_BPW, _E_DIM), jnp.float32),
        pltpu.VMEM((_BPW,), jnp.float32),
        pltpu.VMEM((_SL,), jnp.float32),
        pltpu.VMEM_SHARED((_N_E,), jnp.float32),
        pltpu.SemaphoreType.DMA,
    ],
)
def _sc_gather(table_hbm, idx_hbm, out_hbm, cnt_hbm,
               idx_v, rows_v, ones_v, zero_v, hist_sh, sem):
    c = lax.axis_index("c")
    s = lax.axis_index("s")
    wid = s * _NC + c
    base = wid * _BPW
    pltpu.sync_copy(idx_hbm.at[pl.ds(base, _BPW)], idx_v)
    pltpu.async_copy(table_hbm.at[idx_v], rows_v, sem).wait()
    pltpu.sync_copy(rows_v, out_hbm.at[pl.ds(base, _BPW)])

    for k in range(_BPW // 16):
        ones_v[pl.ds(k * 16, 16)] = jnp.full((16,), 1.0, jnp.float32)
    for k in range(_SL // 16):
        zero_v[pl.ds(k * 16, 16)] = jnp.zeros((16,), jnp.float32)
    pltpu.sync_copy(zero_v, hist_sh.at[pl.ds(s * _SL, _SL)])
    plsc.subcore_barrier()
    pltpu.sync_copy(ones_v, hist_sh.at[idx_v], add=True)
    plsc.subcore_barrier()
    pltpu.sync_copy(hist_sh.at[pl.ds(s * _SL, _SL)],
                    cnt_hbm.at[c, pl.ds(s * _SL, _SL)])


def kernel(z, embedding_weight):
    zf = z.reshape(-1, _E_DIM)
    en = _en_call(embedding_weight)
    idx, oh, dmin = _vq_call(zf, embedding_weight, en)
    zq, cnt2 = _sc_gather(embedding_weight, idx.reshape(_M))
    loss, ppl = _fin_call(cnt2, dmin)
    return (loss[0, 0], zq.reshape(z.shape), ppl[0, 0], oh, idx)
